# Initial kernel scaffold; baseline (speedup 1.0000x reference)
import functools

import jax
import jax.numpy as jnp
from jax import lax
from jax.experimental import pallas as pl
from jax.experimental.pallas import tpu as pltpu
from jax.experimental.pallas import tpu_sc as plsc

N = 10000
NP = 10240
E = 320000
D = 128
G = 64
NC = 2
NS = 16
NW = NC * NS
NPW = NP // NW
ACH = 80
EW = E // NW
EC = 80
NCH = EW // EC


def _sc_mesh():
    return plsc.VectorSubcoreMesh(
        core_axis_name="c", subcore_axis_name="s", num_cores=NC, num_subcores=NS
    )


def _encode_body(xT, atomflat, ewT, dst, bond,
                 h_out, ew_out, degp_out,
                 idx_v, rows_v, h_v, bond_v,
                 e0_v, e1_v, e2_v, dst_v, ew_v, degp_v, sem):
    c = lax.axis_index("c")
    s = lax.axis_index("s")
    wid = s * NC + c

    for sub in range(NPW // ACH):
        nbase = wid * NPW + sub * ACH
        for f in range(9):
            pltpu.sync_copy(xT.at[f, pl.ds(nbase, ACH)], idx_v)

            def _off(i, _, f=f):
                v = idx_v[pl.ds(i * 16, 16)]
                idx_v[pl.ds(i * 16, 16)] = v + f * 119
                return _
            lax.fori_loop(0, ACH // 16, _off, None)
            pltpu.async_copy(atomflat.at[idx_v], rows_v, sem).wait()

            def _acc(r, _, f=f):
                for q in range(8):
                    sl = pl.ds(q * 16, 16)
                    if f == 0:
                        h_v[r, sl] = rows_v[r, sl]
                    else:
                        h_v[r, sl] = h_v[r, sl] + rows_v[r, sl]
                return _
            lax.fori_loop(0, ACH, _acc, None)
        pltpu.sync_copy(h_v, h_out.at[pl.ds(nbase, ACH)])

    pltpu.sync_copy(bond, bond_v)

    def _zero_deg(i, _):
        degp_v[pl.ds(i * 16, 16)] = jnp.zeros((16,), jnp.float32)
        return _
    lax.fori_loop(0, NP // 16, _zero_deg, None)

    ECH2 = 2000
    for ch in range(EW // ECH2):
        ebase = wid * EW + ch * ECH2
        pltpu.sync_copy(ewT.at[0, pl.ds(ebase, ECH2)], e0_v)
        pltpu.sync_copy(ewT.at[1, pl.ds(ebase, ECH2)], e1_v)
        pltpu.sync_copy(ewT.at[2, pl.ds(ebase, ECH2)], e2_v)
        pltpu.sync_copy(dst.at[pl.ds(ebase, ECH2)], dst_v)

        def _ew(j, _):
            sl = pl.ds(j * 16, 16)
            i0 = e0_v[sl]
            i1 = e1_v[sl] + 5
            i2 = e2_v[sl] + 10
            g0 = plsc.load_gather(bond_v, [i0])
            g1 = plsc.load_gather(bond_v, [i1])
            g2 = plsc.load_gather(bond_v, [i2])
            w = g0 + g1 + g2 + 1.0
            ew_v[sl] = w
            plsc.addupdate_scatter(degp_v, [dst_v[sl]], w)
            return _
        lax.fori_loop(0, ECH2 // 16, _ew, None)
        pltpu.sync_copy(ew_v, ew_out.at[pl.ds(ebase, ECH2)])

    pltpu.sync_copy(degp_v, degp_out.at[wid])


def _encode(xT, atomflat, ewT, dst, bond):
    body = functools.partial(
        pl.kernel,
        out_type=(
            jax.ShapeDtypeStruct((NP, D), jnp.float32),
            jax.ShapeDtypeStruct((E,), jnp.float32),
            jax.ShapeDtypeStruct((NW, NP), jnp.float32),
        ),
        mesh=_sc_mesh(),
        scratch_types=[
            pltpu.VMEM((ACH,), jnp.int32),
            pltpu.VMEM((ACH, D), jnp.float32),
            pltpu.VMEM((ACH, D), jnp.float32),
            pltpu.VMEM((16,), jnp.float32),
            pltpu.VMEM((2000,), jnp.int32),
            pltpu.VMEM((2000,), jnp.int32),
            pltpu.VMEM((2000,), jnp.int32),
            pltpu.VMEM((2000,), jnp.int32),
            pltpu.VMEM((2000,), jnp.float32),
            pltpu.VMEM((NP,), jnp.float32),
            pltpu.SemaphoreType.DMA,
        ],
    )(_encode_body)
    return body(xT, atomflat, ewT, dst, bond)


def _edge_body(hw, src, dst, ew, dinv,
               accs_out,
               dinv_v, srcb, dstb, ewb, normb, rows, acc_sh, sem):
    c = lax.axis_index("c")
    s = lax.axis_index("s")
    wid = s * NC + c

    pltpu.sync_copy(dinv, dinv_v)

    def _zrow(r, _):
        for q in range(8):
            rows[r, pl.ds(q * 16, 16)] = jnp.zeros((16,), jnp.float32)
        return _
    lax.fori_loop(0, EC, _zrow, None)
    for k in range(NP // NS // EC):
        pltpu.sync_copy(rows, acc_sh.at[pl.ds(s * (NP // NS) + k * EC, EC)])
    plsc.subcore_barrier()

    def _chunk(ch, _):
        ebase = wid * EW + ch * EC
        pltpu.sync_copy(src.at[pl.ds(ebase, EC)], srcb)
        pltpu.sync_copy(dst.at[pl.ds(ebase, EC)], dstb)
        pltpu.sync_copy(ew.at[pl.ds(ebase, EC)], ewb)
        pltpu.async_copy(hw.at[srcb], rows, sem).wait()

        def _norm(j, _):
            sl = pl.ds(j * 16, 16)
            da = plsc.load_gather(dinv_v, [srcb[sl]])
            db = plsc.load_gather(dinv_v, [dstb[sl]])
            normb[sl] = da * ewb[sl] * db
            return _
        lax.fori_loop(0, EC // 16, _norm, None)

        def _scale(e, _):
            nj = normb[e]
            for q in range(8):
                sl = pl.ds(q * 16, 16)
                rows[e, sl] = rows[e, sl] * nj
            return _
        lax.fori_loop(0, EC, _scale, None)

        pltpu.sync_copy(rows, acc_sh.at[dstb], add=True)
        return _
    lax.fori_loop(0, NCH, _chunk, None)

    plsc.subcore_barrier()
    for k in range(NP // NS // EC):
        base = s * (NP // NS) + k * EC
        pltpu.sync_copy(acc_sh.at[pl.ds(base, EC)], rows)
        pltpu.sync_copy(rows, accs_out.at[c, pl.ds(base, EC)])


def _edge_pass(hw, src, dst, ew, dinv):
    body = functools.partial(
        pl.kernel,
        out_type=jax.ShapeDtypeStruct((NC, NP, D), jnp.float32),
        mesh=_sc_mesh(),
        scratch_types=[
            pltpu.VMEM((NP,), jnp.float32),
            pltpu.VMEM((EC,), jnp.int32),
            pltpu.VMEM((EC,), jnp.int32),
            pltpu.VMEM((EC,), jnp.float32),
            pltpu.VMEM((EC,), jnp.float32),
            pltpu.VMEM((EC, D), jnp.float32),
            pltpu.VMEM_SHARED((NP, D), jnp.float32),
            pltpu.SemaphoreType.DMA,
        ],
    )(_edge_body)
    return body(hw, src, dst, ew, dinv)


def _dinv_mm_body(degp_ref, h_ref, w_ref, hw_ref, dinv_ref):
    deg = jnp.sum(degp_ref[...], axis=0) + 1.0
    dinv_ref[...] = jnp.where(deg > 0, lax.rsqrt(deg), 0.0)[None, :]
    hw_ref[...] = jnp.dot(h_ref[...], w_ref[...],
                          preferred_element_type=jnp.float32)


def _dinv_mm(degp, h, W):
    return pl.pallas_call(
        _dinv_mm_body,
        grid=(NP // D,),
        in_specs=[
            pl.BlockSpec((NW, D), lambda i: (0, i)),
            pl.BlockSpec((D, D), lambda i: (i, 0)),
            pl.BlockSpec((D, D), lambda i: (0, 0)),
        ],
        out_specs=[
            pl.BlockSpec((D, D), lambda i: (i, 0)),
            pl.BlockSpec((1, D), lambda i: (i, 0)),
        ],
        out_shape=[
            jax.ShapeDtypeStruct((NP, D), jnp.float32),
            jax.ShapeDtypeStruct((NP // D, D), jnp.float32),
        ],
    )(degp, h, W)


def _combine_mm_body(accs_ref, hw_ref, dinv_ref, b_ref, w_ref, out_ref):
    d = dinv_ref[0, :]
    h2 = (accs_ref[0] + accs_ref[1]
          + (d * d)[:, None] * hw_ref[...] + b_ref[0, :][None, :])
    h2 = jnp.maximum(h2, 0.0)
    out_ref[...] = jnp.dot(h2, w_ref[...], preferred_element_type=jnp.float32)


def _combine_mm(accs, hw, dinv2d, b, W):
    return pl.pallas_call(
        _combine_mm_body,
        grid=(NP // D,),
        in_specs=[
            pl.BlockSpec((NC, D, D), lambda i: (0, i, 0)),
            pl.BlockSpec((D, D), lambda i: (i, 0)),
            pl.BlockSpec((1, D), lambda i: (i, 0)),
            pl.BlockSpec((1, D), lambda i: (0, 0)),
            pl.BlockSpec((D, D), lambda i: (0, 0)),
        ],
        out_specs=pl.BlockSpec((D, D), lambda i: (i, 0)),
        out_shape=jax.ShapeDtypeStruct((NP, D), jnp.float32),
    )(accs, hw, dinv2d, b, W)


def _readout_body(accs_ref, hw_ref, dinv_ref, b_ref, bidx_ref, wl_ref, bl_ref,
                  out_ref, sums_ref, cnt_ref):
    i = pl.program_id(0)

    @pl.when(i == 0)
    def _():
        sums_ref[...] = jnp.zeros_like(sums_ref)
        cnt_ref[...] = jnp.zeros_like(cnt_ref)

    d = dinv_ref[0, :]
    h3 = (accs_ref[0] + accs_ref[1]
          + (d * d)[:, None] * hw_ref[...] + b_ref[0, :][None, :])
    bidx = bidx_ref[0, :]
    onehot = (bidx[:, None] == lax.broadcasted_iota(jnp.int32, (D, G), 1))
    onehot = onehot.astype(jnp.float32)
    dn = (((0,), (0,)), ((), ()))
    sums_ref[...] += lax.dot_general(onehot, h3, dn,
                                     preferred_element_type=jnp.float32)
    cnt_ref[...] += lax.dot_general(onehot, jnp.ones((D, D), jnp.float32), dn,
                                    preferred_element_type=jnp.float32)

    @pl.when(i == NP // D - 1)
    def _():
        mean = sums_ref[...] / jnp.maximum(cnt_ref[...], 1.0)
        z = jnp.sum(mean * wl_ref[0, :][None, :], axis=1, keepdims=True)
        out_ref[...] = jax.nn.sigmoid(z + bl_ref[0, :][None, :])


def _readout(accs, hw, dinv2d, b, bidx2d, wl_row, bl_row):
    return pl.pallas_call(
        _readout_body,
        grid=(NP // D,),
        in_specs=[
            pl.BlockSpec((NC, D, D), lambda i: (0, i, 0)),
            pl.BlockSpec((D, D), lambda i: (i, 0)),
            pl.BlockSpec((1, D), lambda i: (i, 0)),
            pl.BlockSpec((1, D), lambda i: (0, 0)),
            pl.BlockSpec((1, D), lambda i: (i, 0)),
            pl.BlockSpec((1, D), lambda i: (0, 0)),
            pl.BlockSpec((1, D), lambda i: (0, 0)),
        ],
        out_specs=pl.BlockSpec((G, D), lambda i: (0, 0)),
        out_shape=jax.ShapeDtypeStruct((G, D), jnp.float32),
        scratch_shapes=[
            pltpu.VMEM((G, D), jnp.float32),
            pltpu.VMEM((G, D), jnp.float32),
        ],
    )(accs, hw, dinv2d, b, bidx2d, wl_row, bl_row)


def kernel(x, edge_index, edge_weight, batch_idx, atom_emb, bond_emb,
           W1, b1, W2, b2, Wl, bl):
    xT = jnp.pad(x.astype(jnp.int32), ((0, NP - N), (0, 0))).T
    atomflat = atom_emb.reshape(9 * 119, D)
    ewT = edge_weight.astype(jnp.int32).T
    src = edge_index[0].astype(jnp.int32)
    dst = edge_index[1].astype(jnp.int32)
    bond = jnp.pad(bond_emb.reshape(15), (0, 1))
    bidx2d = jnp.pad(batch_idx.astype(jnp.int32), (0, NP - N),
                     constant_values=G).reshape(NP // D, D)

    h, ew, degp = _encode(xT, atomflat, ewT, dst, bond)
    hw1, dinv2d = _dinv_mm(degp, h, W1)
    dinv = dinv2d.reshape(NP)
    accs1 = _edge_pass(hw1, src, dst, ew, dinv)
    hw2 = _combine_mm(accs1, hw1, dinv2d, b1.reshape(1, D), W2)
    accs2 = _edge_pass(hw2, src, dst, ew, dinv)
    out_p = _readout(accs2, hw2, dinv2d, b2.reshape(1, D), bidx2d,
                     Wl.reshape(1, D), jnp.broadcast_to(bl[None], (1, D)))
    return out_p[:, :1]


# speedup vs baseline: 7.7257x; 7.7257x over previous
import functools

import jax
import jax.numpy as jnp
from jax import lax
from jax.experimental import pallas as pl
from jax.experimental.pallas import tpu as pltpu
from jax.experimental.pallas import tpu_sc as plsc

N = 10000
NP = 10240
E = 320000
D = 128
G = 64
NC = 2
NS = 16
NW = NC * NS
NPW = NP // NW
ACH = 80
EW = E // NW
EC = 80
NCH = EW // EC


def _sc_mesh():
    return plsc.VectorSubcoreMesh(
        core_axis_name="c", subcore_axis_name="s", num_cores=NC, num_subcores=NS
    )


def _encode_body(xF, atomflat, ewF, dst, bond,
                 h_out, ew_out, degp_out,
                 idx_v, rows_v, h_v, bond_v,
                 e0_v, e1_v, e2_v, dst_v, ew_v, degp_v, sem):
    c = lax.axis_index("c")
    s = lax.axis_index("s")
    wid = s * NC + c

    for sub in range(NPW // ACH):
        nbase = wid * NPW + sub * ACH
        for f in range(9):
            pltpu.sync_copy(xF.at[pl.ds(f * NP + nbase, ACH)], idx_v)

            def _off(i, _, f=f):
                v = idx_v[pl.ds(i * 16, 16)]
                idx_v[pl.ds(i * 16, 16)] = v + f * 119
                return _
            lax.fori_loop(0, ACH // 16, _off, None)
            pltpu.async_copy(atomflat.at[idx_v], rows_v, sem).wait()

            def _acc(r, _, f=f):
                for q in range(8):
                    sl = pl.ds(q * 16, 16)
                    if f == 0:
                        h_v[r, sl] = rows_v[r, sl]
                    else:
                        h_v[r, sl] = h_v[r, sl] + rows_v[r, sl]
                return _
            lax.fori_loop(0, ACH, _acc, None)
        pltpu.sync_copy(h_v, h_out.at[pl.ds(nbase, ACH)])

    pltpu.sync_copy(bond, bond_v)

    def _zero_deg(i, _):
        degp_v[pl.ds(i * 16, 16)] = jnp.zeros((16,), jnp.float32)
        return _
    lax.fori_loop(0, NP // 16, _zero_deg, None)

    ECH2 = 2000
    for ch in range(EW // ECH2):
        ebase = wid * EW + ch * ECH2
        pltpu.sync_copy(ewF.at[pl.ds(ebase, ECH2)], e0_v)
        pltpu.sync_copy(ewF.at[pl.ds(E + ebase, ECH2)], e1_v)
        pltpu.sync_copy(ewF.at[pl.ds(2 * E + ebase, ECH2)], e2_v)
        pltpu.sync_copy(dst.at[pl.ds(ebase, ECH2)], dst_v)

        def _ew(j, _):
            sl = pl.ds(j * 16, 16)
            i0 = e0_v[sl]
            i1 = e1_v[sl] + 5
            i2 = e2_v[sl] + 10
            g0 = plsc.load_gather(bond_v, [i0])
            g1 = plsc.load_gather(bond_v, [i1])
            g2 = plsc.load_gather(bond_v, [i2])
            w = g0 + g1 + g2 + 1.0
            ew_v[sl] = w
            plsc.addupdate_scatter(degp_v, [dst_v[sl]], w)
            return _
        lax.fori_loop(0, ECH2 // 16, _ew, None)
        pltpu.sync_copy(ew_v, ew_out.at[pl.ds(ebase, ECH2)])

    pltpu.sync_copy(degp_v, degp_out.at[wid])


def _encode(xF, atomflat, ewF, dst, bond):
    body = functools.partial(
        pl.kernel,
        out_type=(
            jax.ShapeDtypeStruct((NP, D), jnp.float32),
            jax.ShapeDtypeStruct((E,), jnp.float32),
            jax.ShapeDtypeStruct((NW, NP), jnp.float32),
        ),
        mesh=_sc_mesh(),
        scratch_types=[
            pltpu.VMEM((ACH,), jnp.int32),
            pltpu.VMEM((ACH, D), jnp.float32),
            pltpu.VMEM((ACH, D), jnp.float32),
            pltpu.VMEM((16,), jnp.float32),
            pltpu.VMEM((2000,), jnp.int32),
            pltpu.VMEM((2000,), jnp.int32),
            pltpu.VMEM((2000,), jnp.int32),
            pltpu.VMEM((2000,), jnp.int32),
            pltpu.VMEM((2000,), jnp.float32),
            pltpu.VMEM((NP,), jnp.float32),
            pltpu.SemaphoreType.DMA,
        ],
        compiler_params=pltpu.CompilerParams(needs_layout_passes=False),
    )(_encode_body)
    return body(xF, atomflat, ewF, dst, bond)


def _edge_body(hw, src, dst, ew, dinv,
               accs_out,
               dinv_v, srcb, dstb, ewb, rows, acc_sh, sem):
    c = lax.axis_index("c")
    s = lax.axis_index("s")
    wid = s * NC + c

    pltpu.sync_copy(dinv, dinv_v)

    def _zrow(r, _):
        for q in range(8):
            rows[r, pl.ds(q * 16, 16)] = jnp.zeros((16,), jnp.float32)
        return _
    lax.fori_loop(0, EC, _zrow, None)
    for k in range(NP // NS // EC):
        pltpu.sync_copy(rows, acc_sh.at[pl.ds(s * (NP // NS) + k * EC, EC)])
    plsc.subcore_barrier()

    def _chunk(ch, _):
        ebase = wid * EW + ch * EC
        pltpu.sync_copy(src.at[pl.ds(ebase, EC)], srcb)
        pltpu.sync_copy(dst.at[pl.ds(ebase, EC)], dstb)
        pltpu.sync_copy(ew.at[pl.ds(ebase, EC)], ewb)
        pltpu.async_copy(hw.at[srcb], rows, sem).wait()

        def _scale(j, _):
            sl = pl.ds(j * 16, 16)
            da = plsc.load_gather(dinv_v, [srcb[sl]])
            db = plsc.load_gather(dinv_v, [dstb[sl]])
            nv = da * ewb[sl] * db
            for jj in range(16):
                nj = nv[jj]
                e = j * 16 + jj
                for q in range(8):
                    s2 = pl.ds(q * 16, 16)
                    rows[e, s2] = rows[e, s2] * nj
            return _
        lax.fori_loop(0, EC // 16, _scale, None)

        pltpu.sync_copy(rows, acc_sh.at[dstb], add=True)
        return _
    lax.fori_loop(0, NCH, _chunk, None)

    plsc.subcore_barrier()
    for k in range(NP // NS // EC):
        base = s * (NP // NS) + k * EC
        pltpu.sync_copy(acc_sh.at[pl.ds(base, EC)], rows)
        pltpu.sync_copy(rows, accs_out.at[c, pl.ds(base, EC)])


def _edge_pass(hw, src, dst, ew, dinv):
    body = functools.partial(
        pl.kernel,
        out_type=jax.ShapeDtypeStruct((NC, NP, D), jnp.float32),
        mesh=_sc_mesh(),
        scratch_types=[
            pltpu.VMEM((NP,), jnp.float32),
            pltpu.VMEM((EC,), jnp.int32),
            pltpu.VMEM((EC,), jnp.int32),
            pltpu.VMEM((EC,), jnp.float32),
            pltpu.VMEM((EC, D), jnp.float32),
            pltpu.VMEM_SHARED((NP, D), jnp.float32),
            pltpu.SemaphoreType.DMA,
        ],
        compiler_params=pltpu.CompilerParams(needs_layout_passes=False),
    )(_edge_body)
    return body(hw, src, dst, ew, dinv)


def _dinv_mm_body(degp_ref, h_ref, w_ref, hw_ref, dinv_ref):
    deg = jnp.sum(degp_ref[...], axis=0) + 1.0
    dinv_ref[...] = jnp.where(deg > 0, lax.rsqrt(deg), 0.0)[None, None, :]
    hw_ref[...] = jnp.dot(h_ref[...], w_ref[...],
                          preferred_element_type=jnp.float32)


def _dinv_mm(degp, h, W):
    return pl.pallas_call(
        _dinv_mm_body,
        grid=(NP // D,),
        in_specs=[
            pl.BlockSpec((NW, D), lambda i: (0, i)),
            pl.BlockSpec((D, D), lambda i: (i, 0)),
            pl.BlockSpec((D, D), lambda i: (0, 0)),
        ],
        out_specs=[
            pl.BlockSpec((D, D), lambda i: (i, 0)),
            pl.BlockSpec((1, 1, D), lambda i: (i, 0, 0)),
        ],
        out_shape=[
            jax.ShapeDtypeStruct((NP, D), jnp.float32),
            jax.ShapeDtypeStruct((NP // D, 1, D), jnp.float32),
        ],
    )(degp, h, W)


def _combine_mm_body(accs_ref, hw_ref, dinv_ref, b_ref, w_ref, out_ref):
    d = dinv_ref[0, 0, :]
    h2 = (accs_ref[0] + accs_ref[1]
          + (d * d)[:, None] * hw_ref[...] + b_ref[0, :][None, :])
    h2 = jnp.maximum(h2, 0.0)
    out_ref[...] = jnp.dot(h2, w_ref[...], preferred_element_type=jnp.float32)


def _combine_mm(accs, hw, dinv2d, b, W):
    return pl.pallas_call(
        _combine_mm_body,
        grid=(NP // D,),
        in_specs=[
            pl.BlockSpec((NC, D, D), lambda i: (0, i, 0)),
            pl.BlockSpec((D, D), lambda i: (i, 0)),
            pl.BlockSpec((1, 1, D), lambda i: (i, 0, 0)),
            pl.BlockSpec((1, D), lambda i: (0, 0)),
            pl.BlockSpec((D, D), lambda i: (0, 0)),
        ],
        out_specs=pl.BlockSpec((D, D), lambda i: (i, 0)),
        out_shape=jax.ShapeDtypeStruct((NP, D), jnp.float32),
    )(accs, hw, dinv2d, b, W)


def _readout_body(accs_ref, hw_ref, dinv_ref, b_ref, bidx_ref, wl_ref, bl_ref,
                  out_ref, sums_ref, cnt_ref):
    i = pl.program_id(0)

    @pl.when(i == 0)
    def _():
        sums_ref[...] = jnp.zeros_like(sums_ref)
        cnt_ref[...] = jnp.zeros_like(cnt_ref)

    d = dinv_ref[0, 0, :]
    h3 = (accs_ref[0] + accs_ref[1]
          + (d * d)[:, None] * hw_ref[...] + b_ref[0, :][None, :])
    bidx = bidx_ref[0, 0, :]
    onehot = (bidx[:, None] == lax.broadcasted_iota(jnp.int32, (D, G), 1))
    onehot = onehot.astype(jnp.float32)
    dn = (((0,), (0,)), ((), ()))
    sums_ref[...] += lax.dot_general(onehot, h3, dn,
                                     preferred_element_type=jnp.float32)
    cnt_ref[...] += lax.dot_general(onehot, jnp.ones((D, D), jnp.float32), dn,
                                    preferred_element_type=jnp.float32)

    @pl.when(i == NP // D - 1)
    def _():
        mean = sums_ref[...] / jnp.maximum(cnt_ref[...], 1.0)
        z = jnp.sum(mean * wl_ref[0, :][None, :], axis=1, keepdims=True)
        out_ref[...] = jax.nn.sigmoid(z + bl_ref[0, :][None, :])


def _readout(accs, hw, dinv2d, b, bidx2d, wl_row, bl_row):
    return pl.pallas_call(
        _readout_body,
        grid=(NP // D,),
        in_specs=[
            pl.BlockSpec((NC, D, D), lambda i: (0, i, 0)),
            pl.BlockSpec((D, D), lambda i: (i, 0)),
            pl.BlockSpec((1, 1, D), lambda i: (i, 0, 0)),
            pl.BlockSpec((1, D), lambda i: (0, 0)),
            pl.BlockSpec((1, 1, D), lambda i: (i, 0, 0)),
            pl.BlockSpec((1, D), lambda i: (0, 0)),
            pl.BlockSpec((1, D), lambda i: (0, 0)),
        ],
        out_specs=pl.BlockSpec((G, D), lambda i: (0, 0)),
        out_shape=jax.ShapeDtypeStruct((G, D), jnp.float32),
        scratch_shapes=[
            pltpu.VMEM((G, D), jnp.float32),
            pltpu.VMEM((G, D), jnp.float32),
        ],
    )(accs, hw, dinv2d, b, bidx2d, wl_row, bl_row)


def kernel(x, edge_index, edge_weight, batch_idx, atom_emb, bond_emb,
           W1, b1, W2, b2, Wl, bl):
    xF = jnp.pad(x.astype(jnp.int32), ((0, NP - N), (0, 0))).T.reshape(9 * NP)
    atomflat = atom_emb.reshape(9 * 119, D)
    ewF = edge_weight.astype(jnp.int32).T.reshape(3 * E)
    src = edge_index[0].astype(jnp.int32)
    dst = edge_index[1].astype(jnp.int32)
    bond = jnp.pad(bond_emb.reshape(15), (0, 1))
    bidx2d = jnp.pad(batch_idx.astype(jnp.int32), (0, NP - N),
                     constant_values=G).reshape(NP // D, 1, D)

    h, ew, degp = _encode(xF, atomflat, ewF, dst, bond)
    hw1, dinv2d = _dinv_mm(degp, h, W1)
    dinv = dinv2d.reshape(NP)
    accs1 = _edge_pass(hw1, src, dst, ew, dinv)
    hw2 = _combine_mm(accs1, hw1, dinv2d, b1.reshape(1, D), W2)
    accs2 = _edge_pass(hw2, src, dst, ew, dinv)
    out_p = _readout(accs2, hw2, dinv2d, b2.reshape(1, D), bidx2d,
                     Wl.reshape(1, D), jnp.broadcast_to(bl[None], (1, D)))
    return out_p[:, :1]
